# baseline (speedup 1.0000x reference)
import functools

import jax
import jax.numpy as jnp
from jax import lax
from jax.experimental import pallas as pl
from jax.experimental.pallas import tpu as pltpu
from jax.experimental.pallas import tpu_sc as plsc

B = 4096
L = 200
C = 64
LH = L // 2

_info = plsc.get_sparse_core_info()
NC = _info.num_cores
NS = _info.num_subcores
NW = NC * NS
NB = B // NW


def _body(x_hbm, w_hbm, out_hbm, idx_v, rows_v, outt_v, sem):
    wid = lax.axis_index("s") * NC + lax.axis_index("c")
    base = wid * NB

    iota = lax.iota(jnp.int32, 16)
    col_base = [(iota + cb * 16) * L for cb in range(4)]

    def per_b(bi, carry):
        b = base + bi
        pltpu.sync_copy(x_hbm.at[b], idx_v)
        cp0 = pltpu.async_copy(w_hbm.at[idx_v.at[0]], rows_v.at[0], sem)
        cp1 = pltpu.async_copy(w_hbm.at[idx_v.at[1]], rows_v.at[1], sem)
        cp0.wait()
        cp1.wait()
        for h in range(2):
            def per_j(j, c2, h=h):
                l = h * LH + j
                for cb in range(4):
                    v = rows_v[h, j, pl.ds(cb * 16, 16)]
                    plsc.store_scatter(outt_v, [col_base[cb] + l], v)
                return c2
            lax.fori_loop(0, LH, per_j, 0)
        pltpu.sync_copy(outt_v, out_hbm.at[b])
        return carry

    lax.fori_loop(0, NB, per_b, 0)


@functools.partial(jax.jit, static_argnames=())
def _sc_embed(x3, w):
    mesh = plsc.VectorSubcoreMesh(core_axis_name="c", subcore_axis_name="s")
    f = pl.kernel(
        _body,
        mesh=mesh,
        out_type=jax.ShapeDtypeStruct((B, C * L), jnp.float32),
        scratch_types=[
            pltpu.VMEM((2, LH), jnp.int32),
            pltpu.VMEM((2, LH, C), jnp.float32),
            pltpu.VMEM((C * L,), jnp.float32),
            pltpu.SemaphoreType.DMA,
        ],
        compiler_params=pltpu.CompilerParams(
            needs_layout_passes=False, use_tc_tiling_on_sc=False),
    )
    return f(x3, w)


def kernel(x, W):
    x3 = x.astype(jnp.int32).reshape(B, 2, LH)
    return _sc_embed(x3, W).reshape(B, C, L)


# speedup vs baseline: 1.2178x; 1.2178x over previous
import functools

import jax
import jax.numpy as jnp
from jax import lax
from jax.experimental import pallas as pl
from jax.experimental.pallas import tpu as pltpu
from jax.experimental.pallas import tpu_sc as plsc

B = 4096
L = 200
C = 64
LH = L // 2

_info = plsc.get_sparse_core_info()
NC = _info.num_cores
NS = _info.num_subcores
NW = NC * NS
NB = B // NW
G = 2
NG = NB // G
NP = NG // 2


def _body(x_hbm, w_hbm, out_hbm, idx_v, rows_v, outt_v, gsem, osem):
    wid = lax.axis_index("s") * NC + lax.axis_index("c")
    base = wid * NB

    iota = lax.iota(jnp.int32, 16)
    col_base = [(iota + cb * 16) * L for cb in range(4)]

    def start_gather(slot, g):
        pltpu.async_copy(
            w_hbm.at[idx_v.at[pl.ds(g * G * L, G * L)]],
            rows_v.at[slot], gsem.at[slot])

    def wait_gather(slot):
        pltpu.make_async_copy(
            w_hbm.at[idx_v.at[pl.ds(0, G * L)]],
            rows_v.at[slot], gsem.at[slot]).wait()

    def start_out(o, b):
        pltpu.async_copy(outt_v.at[o], out_hbm.at[b], osem.at[o])

    def wait_out(o):
        pltpu.make_async_copy(outt_v.at[o], out_hbm.at[0], osem.at[o]).wait()

    def transpose_b(slot, k, o):
        def per_l(l, c2):
            for cb in range(4):
                v = rows_v[slot, k * L + l, pl.ds(cb * 16, 16)]
                plsc.store_scatter(outt_v.at[o], [col_base[cb] + l], v)
            return c2
        lax.fori_loop(0, L, per_l, 0)

    pltpu.sync_copy(x_hbm.at[pl.ds(wid * NB * L, NB * L)], idx_v)
    start_gather(0, 0)

    def per_pair(gp, carry):
        start_gather(1, 2 * gp + 1)
        wait_gather(0)
        for k in range(G):
            b_loc = 2 * G * gp + k
            @pl.when(gp > 0)
            def _():
                wait_out(k)
            transpose_b(0, k, k)
            start_out(k, base + b_loc)
        @pl.when(gp < NP - 1)
        def _():
            start_gather(0, 2 * gp + 2)
        wait_gather(1)
        for k in range(G):
            b_loc = 2 * G * gp + G + k
            @pl.when(gp > 0)
            def _():
                wait_out(G + k)
            transpose_b(1, k, G + k)
            start_out(G + k, base + b_loc)
        return carry

    lax.fori_loop(0, NP, per_pair, 0)
    for o in range(2 * G):
        wait_out(o)


@functools.partial(jax.jit, static_argnames=())
def _sc_embed(x2, w):
    mesh = plsc.VectorSubcoreMesh(core_axis_name="c", subcore_axis_name="s")
    f = pl.kernel(
        _body,
        mesh=mesh,
        out_type=jax.ShapeDtypeStruct((B, C * L), jnp.float32),
        scratch_types=[
            pltpu.VMEM((NB * L,), jnp.int32),
            pltpu.VMEM((2, G * L, C), jnp.float32),
            pltpu.VMEM((2 * G, C * L), jnp.float32),
            pltpu.SemaphoreType.DMA((2,)),
            pltpu.SemaphoreType.DMA((2 * G,)),
        ],
        compiler_params=pltpu.CompilerParams(
            needs_layout_passes=False, use_tc_tiling_on_sc=False),
    )
    return f(x2, w)


def kernel(x, W):
    x2 = x.astype(jnp.int32).reshape(B * L)
    return _sc_embed(x2, W).reshape(B, C, L)
